# Initial kernel scaffold; baseline (speedup 1.0000x reference)
import jax
import jax.numpy as jnp
from jax.experimental import pallas as pl


def kernel(feat, edge_index, edge_weight, W, b):
    raise NotImplementedError("write your pallas kernel here")


# speedup vs baseline: 5.8509x; 5.8509x over previous
import functools

import jax
import jax.numpy as jnp
from jax import lax
from jax.experimental import pallas as pl
from jax.experimental.pallas import tpu as pltpu
from jax.experimental.pallas import tpu_sc as plsc

N_NODES = 10000
N_EDGES = 320000
F = 128

NC = 2
NS = 16
L = 16

N_PAD = 10240
CHUNK = 128
CPT = 80
BLK = 8
E_PAD = NC * NS * CPT * CHUNK
ROWS_PER_TILE = N_PAD // NS


def _rsqrt_newton(d):
    s = d
    for _ in range(14):
        s = 0.5 * (s + d / s)
    return 1.0 / s


def _sc_body(src_hbm, dst_hbm, ew_hbm, feat_hbm, acc_out,
             acc_sh, hist_s_sh, hist_d_sh, norm_s_sh, norm_d_sh,
             sblk, dblk, eblk, norm_s_t, norm_d_t,
             rows_t, w_t, ones_t, nbuf):
    core = lax.axis_index("c")
    sub = lax.axis_index("s")

    zero16 = jnp.zeros((L,), jnp.float32)
    one16 = jnp.ones((L,), jnp.float32)

    def _zrow(r, _):
        for q in range(F // L):
            rows_t.at[r][pl.ds(q * L, L)] = zero16
        return _
    lax.fori_loop(0, CHUNK, _zrow, None)
    for q in range(CHUNK // L):
        ones_t[pl.ds(q * L, L)] = one16
        w_t[pl.ds(q * L, L)] = zero16
    for q in range(ROWS_PER_TILE // L):
        nbuf[pl.ds(q * L, L)] = zero16

    row0 = sub * ROWS_PER_TILE
    for k in range(ROWS_PER_TILE // CHUNK):
        pltpu.sync_copy(rows_t, acc_sh.at[pl.ds(row0 + k * CHUNK, CHUNK)])
    pltpu.sync_copy(nbuf, hist_s_sh.at[pl.ds(row0, ROWS_PER_TILE)])
    pltpu.sync_copy(nbuf, hist_d_sh.at[pl.ds(row0, ROWS_PER_TILE)])
    plsc.subcore_barrier()

    def _hist_half(base):
        def _blk(bi, _):
            g = base + bi * BLK
            pltpu.sync_copy(src_hbm.at[pl.ds(g, BLK)], sblk)
            pltpu.sync_copy(dst_hbm.at[pl.ds(g, BLK)], dblk)

            def _hist(c, _c):
                pltpu.sync_copy(ones_t, hist_s_sh.at[sblk.at[c]], add=True)
                pltpu.sync_copy(ones_t, hist_d_sh.at[dblk.at[c]], add=True)
                return _c
            lax.fori_loop(0, BLK, _hist, None)
            return _
        lax.fori_loop(0, CPT // BLK, _blk, None)

    _hist_half(sub * CPT)
    _hist_half((NS + sub) * CPT)
    plsc.subcore_barrier()

    pltpu.sync_copy(hist_s_sh.at[pl.ds(row0, ROWS_PER_TILE)], nbuf)

    def _norm(g, _):
        d = jnp.maximum(nbuf[pl.ds(g * L, L)], 1.0)
        nbuf[pl.ds(g * L, L)] = _rsqrt_newton(d)
        return _
    lax.fori_loop(0, ROWS_PER_TILE // L, _norm, None)
    pltpu.sync_copy(nbuf, norm_s_sh.at[pl.ds(row0, ROWS_PER_TILE)])

    pltpu.sync_copy(hist_d_sh.at[pl.ds(row0, ROWS_PER_TILE)], nbuf)
    lax.fori_loop(0, ROWS_PER_TILE // L, _norm, None)
    pltpu.sync_copy(nbuf, norm_d_sh.at[pl.ds(row0, ROWS_PER_TILE)])
    plsc.subcore_barrier()

    pltpu.sync_copy(norm_s_sh, norm_s_t)
    pltpu.sync_copy(norm_d_sh, norm_d_t)

    base3 = (core * NS + sub) * CPT

    def _blk3(bi, _):
        g = base3 + bi * BLK
        pltpu.sync_copy(src_hbm.at[pl.ds(g, BLK)], sblk)
        pltpu.sync_copy(dst_hbm.at[pl.ds(g, BLK)], dblk)
        pltpu.sync_copy(ew_hbm.at[pl.ds(g, BLK)], eblk)

        def _chunk(c, _c):
            sv_row = sblk.at[c]
            dv_row = dblk.at[c]
            pltpu.sync_copy(feat_hbm.at[sv_row], rows_t)
            for q in range(CHUNK // L):
                sv = sv_row[pl.ds(q * L, L)]
                dv = dv_row[pl.ds(q * L, L)]
                ev = eblk.at[c][pl.ds(q * L, L)]
                wv = ev * plsc.load_gather(norm_s_t, [sv]) \
                        * plsc.load_gather(norm_d_t, [dv])
                w_t[pl.ds(q * L, L)] = wv

            def _scale(r, _r):
                w = plsc.load_gather(w_t, [jnp.full((L,), r, jnp.int32)])
                rv = rows_t.at[r]
                for q in range(F // L):
                    rv[pl.ds(q * L, L)] = rv[pl.ds(q * L, L)] * w
                return _r
            lax.fori_loop(0, CHUNK, _scale, None)
            pltpu.sync_copy(rows_t, acc_sh.at[dv_row], add=True)
            return _c
        lax.fori_loop(0, BLK, _chunk, None)
        return _
    lax.fori_loop(0, CPT // BLK, _blk3, None)
    plsc.subcore_barrier()

    pltpu.sync_copy(acc_sh.at[pl.ds(row0, ROWS_PER_TILE)],
                    acc_out.at[core, pl.ds(row0, ROWS_PER_TILE)])


def _sc_aggregate(src_p, dst_p, ew_p, feat_p):
    mesh = plsc.VectorSubcoreMesh(core_axis_name="c", subcore_axis_name="s")
    return pl.kernel(
        _sc_body,
        out_type=jax.ShapeDtypeStruct((NC, N_PAD, F), jnp.float32),
        mesh=mesh,
        compiler_params=pltpu.CompilerParams(needs_layout_passes=False),
        scratch_types=[
            pltpu.VMEM_SHARED((N_PAD, F), jnp.float32),
            pltpu.VMEM_SHARED((N_PAD,), jnp.float32),
            pltpu.VMEM_SHARED((N_PAD,), jnp.float32),
            pltpu.VMEM_SHARED((N_PAD,), jnp.float32),
            pltpu.VMEM_SHARED((N_PAD,), jnp.float32),
            pltpu.VMEM((BLK, CHUNK), jnp.int32),
            pltpu.VMEM((BLK, CHUNK), jnp.int32),
            pltpu.VMEM((BLK, CHUNK), jnp.float32),
            pltpu.VMEM((N_PAD,), jnp.float32),
            pltpu.VMEM((N_PAD,), jnp.float32),
            pltpu.VMEM((CHUNK, F), jnp.float32),
            pltpu.VMEM((CHUNK,), jnp.float32),
            pltpu.VMEM((CHUNK,), jnp.float32),
            pltpu.VMEM((ROWS_PER_TILE,), jnp.float32),
        ],
    )(src_p, dst_p, ew_p, feat_p)


def _tc_body(acc_ref, w_ref, b_ref, out_ref):
    p = acc_ref[0] + acc_ref[1]
    y = jnp.dot(p, w_ref[...], preferred_element_type=jnp.float32)
    out_ref[...] = y + b_ref[...]


def _tc_matmul(acc, W, b2):
    blk = 1024
    grid = (N_PAD // blk,)
    return pl.pallas_call(
        _tc_body,
        grid=grid,
        in_specs=[
            pl.BlockSpec((NC, blk, F), lambda i: (0, i, 0)),
            pl.BlockSpec((F, F), lambda i: (0, 0)),
            pl.BlockSpec((1, F), lambda i: (0, 0)),
        ],
        out_specs=pl.BlockSpec((blk, F), lambda i: (i, 0)),
        out_shape=jax.ShapeDtypeStruct((N_PAD, F), jnp.float32),
    )(acc, W, b2)


@jax.jit
def kernel(feat, edge_index, edge_weight, W, b):
    src = edge_index[0].astype(jnp.int32)
    dst = edge_index[1].astype(jnp.int32)
    e = src.shape[0]
    npad = E_PAD - e
    pad_idx = (jnp.arange(npad, dtype=jnp.int32) % (N_PAD - N_NODES)) + N_NODES
    src_p = jnp.concatenate([src, pad_idx]).reshape(E_PAD // CHUNK, CHUNK)
    dst_p = jnp.concatenate([dst, pad_idx]).reshape(E_PAD // CHUNK, CHUNK)
    ew_p = jnp.concatenate(
        [edge_weight, jnp.zeros((npad,), jnp.float32)]
    ).reshape(E_PAD // CHUNK, CHUNK)
    feat_p = jnp.pad(feat, ((0, N_PAD - feat.shape[0]), (0, 0)))
    acc = _sc_aggregate(src_p, dst_p, ew_p, feat_p)
    out = _tc_matmul(acc, W, b.reshape(1, F))
    return out[:N_NODES]


# speedup vs baseline: 7.5645x; 1.2929x over previous
import functools

import jax
import jax.numpy as jnp
from jax import lax
from jax.experimental import pallas as pl
from jax.experimental.pallas import tpu as pltpu
from jax.experimental.pallas import tpu_sc as plsc

N_NODES = 10000
N_EDGES = 320000
F = 128

NC = 2
NS = 16
L = 16

N_PAD = 10240
CHUNK = 48
BLK = 8
CPT = 224
NBLK = CPT // BLK
E_PAD = NC * NS * CPT * CHUNK
ROWS_PER_TILE = N_PAD // NS
NBUF = 4
ZB = 40


def _rsqrt_newton(d):
    s = d
    for _ in range(14):
        s = 0.5 * (s + d / s)
    return 1.0 / s


def _bcast(ref, i):
    return plsc.load_gather(ref, [jnp.full((L,), i, jnp.int32)])


def _sc_body(src_hbm, dst_hbm, ew_hbm, feat_hbm, acc_out,
             acc_sh, hist_s_sh, hist_d_sh,
             sblk, dblk, eblk, ones_t, norm_s_t, nd_t,
             rows, w_t, gsem, ssem, hsem):
    core = lax.axis_index("c")
    sub = lax.axis_index("s")
    zero16 = jnp.zeros((L,), jnp.float32)
    one16 = jnp.ones((L,), jnp.float32)
    row0 = sub * ROWS_PER_TILE

    def _zrow(r, _):
        for q in range(F // L):
            rows.at[0, r][pl.ds(q * L, L)] = zero16
        return _
    lax.fori_loop(0, CHUNK, _zrow, None)
    for q in range(CHUNK // L):
        ones_t[pl.ds(q * L, L)] = one16
    for q in range(ROWS_PER_TILE // L):
        nd_t[pl.ds(q * L, L)] = zero16

    for k in range(ROWS_PER_TILE // ZB):
        pltpu.sync_copy(rows.at[0, pl.ds(0, ZB)],
                        acc_sh.at[pl.ds(row0 + k * ZB, ZB)])
    pltpu.sync_copy(nd_t, hist_s_sh.at[pl.ds(row0, ROWS_PER_TILE)])
    pltpu.sync_copy(nd_t, hist_d_sh.at[pl.ds(row0, ROWS_PER_TILE)])
    plsc.subcore_barrier()

    def _h1(k, _):
        p = k % 2
        base = jnp.where(k < NBLK, sub * CPT + k * BLK,
                         (NS + sub) * CPT + (k - NBLK) * BLK)
        pltpu.sync_copy(src_hbm.at[pl.ds(base, BLK)], sblk.at[p])
        pltpu.sync_copy(dst_hbm.at[pl.ds(base, BLK)], dblk.at[p])

        @pl.when(k >= 1)
        def _drain():
            for _i in range(2 * BLK):
                pltpu.make_async_copy(ones_t, hist_s_sh.at[sblk.at[0, 0]],
                                      hsem).wait()

        for i in range(BLK):
            pltpu.async_copy(ones_t, hist_s_sh.at[sblk.at[p, i]],
                             hsem, add=True)
            pltpu.async_copy(ones_t, hist_d_sh.at[dblk.at[p, i]],
                             hsem, add=True)
        return _
    lax.fori_loop(0, 2 * NBLK, _h1, None)
    for _i in range(2 * BLK):
        pltpu.make_async_copy(ones_t, hist_s_sh.at[sblk.at[0, 0]],
                              hsem).wait()
    plsc.subcore_barrier()

    def _norm(ref):
        def _n(g, _):
            d = jnp.maximum(ref[pl.ds(g * L, L)], 1.0)
            ref[pl.ds(g * L, L)] = _rsqrt_newton(d)
            return _
        lax.fori_loop(0, ROWS_PER_TILE // L, _n, None)

    pltpu.sync_copy(hist_s_sh.at[pl.ds(row0, ROWS_PER_TILE)], nd_t)
    _norm(nd_t)
    pltpu.sync_copy(nd_t, hist_s_sh.at[pl.ds(row0, ROWS_PER_TILE)])
    pltpu.sync_copy(hist_d_sh.at[pl.ds(row0, ROWS_PER_TILE)], nd_t)
    _norm(nd_t)
    plsc.subcore_barrier()
    pltpu.sync_copy(hist_s_sh, norm_s_t)

    base3 = (core * NS + sub) * CPT

    def _g_issue(pp, r, buf):
        pltpu.async_copy(feat_hbm.at[sblk.at[pp, r]], rows.at[buf],
                         gsem.at[buf])

    def _g_wait(buf):
        pltpu.make_async_copy(feat_hbm.at[sblk.at[0, 0]], rows.at[buf],
                              gsem.at[buf]).wait()

    def _s_issue(pp, r, buf):
        pltpu.async_copy(rows.at[buf], acc_sh.at[dblk.at[pp, r]],
                         ssem.at[buf], add=True)

    def _s_wait(buf):
        pltpu.make_async_copy(rows.at[buf], acc_sh.at[dblk.at[0, 0]],
                              ssem.at[buf]).wait()

    pltpu.sync_copy(src_hbm.at[pl.ds(base3, BLK)], sblk.at[0])
    pltpu.sync_copy(dst_hbm.at[pl.ds(base3, BLK)], dblk.at[0])
    pltpu.sync_copy(ew_hbm.at[pl.ds(base3, BLK)], eblk.at[0])
    pltpu.sync_copy(src_hbm.at[pl.ds(base3 + BLK, BLK)], sblk.at[1])
    pltpu.sync_copy(dst_hbm.at[pl.ds(base3 + BLK, BLK)], dblk.at[1])
    pltpu.sync_copy(ew_hbm.at[pl.ds(base3 + BLK, BLK)], eblk.at[1])
    for i in range(NBUF - 1):
        _g_issue(0, i, i)

    def _b3(b, _):
        p = b % 2
        for i in range(BLK):
            buf = i % NBUF
            _g_wait(buf)
            for q in range(CHUNK // L):
                sv = sblk.at[p, i][pl.ds(q * L, L)]
                ev = eblk.at[p, i][pl.ds(q * L, L)]
                w_t[pl.ds(q * L, L)] = ev * plsc.load_gather(norm_s_t, [sv])

            def _scale(r2, _c):
                for u in range(2):
                    r = 2 * r2 + u
                    w = _bcast(w_t, r)
                    rv = rows.at[buf, r]
                    for q in range(F // L):
                        rv[pl.ds(q * L, L)] = rv[pl.ds(q * L, L)] * w
                return _c
            lax.fori_loop(0, CHUNK // 2, _scale, None)
            _s_issue(p, i, buf)

            pbuf = (i - 1) % NBUF
            if i == 0:
                @pl.when(b > 0)
                def _w0():
                    _s_wait(pbuf)
            else:
                _s_wait(pbuf)
            if i == 5:
                @pl.when(b + 1 < NBLK)
                def _pref():
                    nb = base3 + (b + 1) * BLK
                    pltpu.sync_copy(src_hbm.at[pl.ds(nb, BLK)],
                                    sblk.at[1 - p])
                    pltpu.sync_copy(dst_hbm.at[pl.ds(nb, BLK)],
                                    dblk.at[1 - p])
                    pltpu.sync_copy(ew_hbm.at[pl.ds(nb, BLK)],
                                    eblk.at[1 - p])
            nxt = b * BLK + i + NBUF - 1

            @pl.when(nxt < CPT)
            def _gi():
                if i + NBUF - 1 < BLK:
                    _g_issue(p, i + NBUF - 1, pbuf)
                else:
                    _g_issue(1 - p, i + NBUF - 1 - BLK, pbuf)
        return _
    lax.fori_loop(0, NBLK, _b3, None)
    _s_wait((CPT - 1) % NBUF)
    plsc.subcore_barrier()

    def _cpo(k, _):
        off = row0 + k * ZB
        pltpu.sync_copy(acc_sh.at[pl.ds(off, ZB)], rows.at[0, pl.ds(0, ZB)])

        def _dsc(r2, _c):
            for u in range(2):
                r = 2 * r2 + u
                w = _bcast(nd_t, k * ZB + r)
                rv = rows.at[0, r]
                for q in range(F // L):
                    rv[pl.ds(q * L, L)] = rv[pl.ds(q * L, L)] * w
            return _c
        lax.fori_loop(0, ZB // 2, _dsc, None)
        pltpu.sync_copy(rows.at[0, pl.ds(0, ZB)],
                        acc_out.at[core, pl.ds(off, ZB)])
        return _
    lax.fori_loop(0, ROWS_PER_TILE // ZB, _cpo, None)


def _sc_aggregate(src_p, dst_p, ew_p, feat_p):
    mesh = plsc.VectorSubcoreMesh(core_axis_name="c", subcore_axis_name="s")
    return pl.kernel(
        _sc_body,
        out_type=jax.ShapeDtypeStruct((NC, N_PAD, F), jnp.float32),
        mesh=mesh,
        compiler_params=pltpu.CompilerParams(needs_layout_passes=False),
        scratch_types=[
            pltpu.VMEM_SHARED((N_PAD, F), jnp.float32),
            pltpu.VMEM_SHARED((N_PAD,), jnp.float32),
            pltpu.VMEM_SHARED((N_PAD,), jnp.float32),
            pltpu.VMEM((2, BLK, CHUNK), jnp.int32),
            pltpu.VMEM((2, BLK, CHUNK), jnp.int32),
            pltpu.VMEM((2, BLK, CHUNK), jnp.float32),
            pltpu.VMEM((CHUNK,), jnp.float32),
            pltpu.VMEM((N_PAD,), jnp.float32),
            pltpu.VMEM((ROWS_PER_TILE,), jnp.float32),
            pltpu.VMEM((NBUF, CHUNK, F), jnp.float32),
            pltpu.VMEM((CHUNK,), jnp.float32),
            pltpu.SemaphoreType.DMA((NBUF,)),
            pltpu.SemaphoreType.DMA((NBUF,)),
            pltpu.SemaphoreType.DMA,
        ],
    )(src_p, dst_p, ew_p, feat_p)


def _tc_body(acc_ref, w_ref, b_ref, out_ref):
    p = acc_ref[0] + acc_ref[1]
    y = jnp.dot(p, w_ref[...], preferred_element_type=jnp.float32)
    out_ref[...] = y + b_ref[...]


def _tc_matmul(acc, W, b2):
    blk = 1024
    grid = (N_PAD // blk,)
    return pl.pallas_call(
        _tc_body,
        grid=grid,
        in_specs=[
            pl.BlockSpec((NC, blk, F), lambda i: (0, i, 0)),
            pl.BlockSpec((F, F), lambda i: (0, 0)),
            pl.BlockSpec((1, F), lambda i: (0, 0)),
        ],
        out_specs=pl.BlockSpec((blk, F), lambda i: (i, 0)),
        out_shape=jax.ShapeDtypeStruct((N_PAD, F), jnp.float32),
    )(acc, W, b2)


@jax.jit
def kernel(feat, edge_index, edge_weight, W, b):
    src = edge_index[0].astype(jnp.int32)
    dst = edge_index[1].astype(jnp.int32)
    e = src.shape[0]
    npad = E_PAD - e
    pad_idx = (jnp.arange(npad, dtype=jnp.int32) % (N_PAD - N_NODES)) + N_NODES
    src_p = jnp.concatenate([src, pad_idx]).reshape(E_PAD // CHUNK, CHUNK)
    dst_p = jnp.concatenate([dst, pad_idx]).reshape(E_PAD // CHUNK, CHUNK)
    ew_p = jnp.concatenate(
        [edge_weight, jnp.zeros((npad,), jnp.float32)]
    ).reshape(E_PAD // CHUNK, CHUNK)
    feat_p = jnp.pad(feat, ((0, N_PAD - feat.shape[0]), (0, 0)))
    acc = _sc_aggregate(src_p, dst_p, ew_p, feat_p)
    out = _tc_matmul(acc, W, b2 := b.reshape(1, F))
    return out[:N_NODES]


# speedup vs baseline: 9.4362x; 1.2474x over previous
import functools

import jax
import jax.numpy as jnp
from jax import lax
from jax.experimental import pallas as pl
from jax.experimental.pallas import tpu as pltpu
from jax.experimental.pallas import tpu_sc as plsc

N_NODES = 10000
N_EDGES = 320000
F = 128

NC = 2
NS = 16
L = 16

N_PAD = 10240
CHUNK = 48
BLK = 8
CPT = 224
NBLK = CPT // BLK
E_PAD = NC * NS * CPT * CHUNK
ROWS_PER_TILE = N_PAD // NS
NBUF = 4
ZB = 40
H1R = E_PAD // 128 // NS


def _rsqrt_newton(d):
    s = d
    for _ in range(14):
        s = 0.5 * (s + d / s)
    return 1.0 / s


def _bcast(ref, i):
    return plsc.load_gather(ref, [jnp.full((L,), i, jnp.int32)])


def _sc_body(src_hbm, dst_hbm, ew_hbm, feat_hbm, src128_hbm,
             acc_out, dh_out,
             acc_sh, hist_s_sh, hist_d_sh,
             sblk, dblk, eblk, sblk1, ones_t, ones_c, norm_s_t, nd_t,
             rows, w_t, gsem, ssem, hsem):
    core = lax.axis_index("c")
    sub = lax.axis_index("s")
    zero16 = jnp.zeros((L,), jnp.float32)
    one16 = jnp.ones((L,), jnp.float32)
    row0 = sub * ROWS_PER_TILE

    def _zrow(r, _):
        for q in range(F // L):
            rows.at[0, r][pl.ds(q * L, L)] = zero16
        return _
    lax.fori_loop(0, CHUNK, _zrow, None)
    for q in range(128 // L):
        ones_t[pl.ds(q * L, L)] = one16
    for q in range(CHUNK // L):
        ones_c[pl.ds(q * L, L)] = one16
    for q in range(ROWS_PER_TILE // L):
        nd_t[pl.ds(q * L, L)] = zero16

    for k in range(ROWS_PER_TILE // ZB):
        pltpu.async_copy(rows.at[0, pl.ds(0, ZB)],
                         acc_sh.at[pl.ds(row0 + k * ZB, ZB)], hsem)
    pltpu.sync_copy(nd_t, hist_s_sh.at[pl.ds(row0, ROWS_PER_TILE)])
    pltpu.sync_copy(nd_t, hist_d_sh.at[pl.ds(row0, ROWS_PER_TILE)])
    for k in range(ROWS_PER_TILE // ZB):
        pltpu.make_async_copy(rows.at[0, pl.ds(0, ZB)],
                              acc_sh.at[pl.ds(row0 + k * ZB, ZB)],
                              hsem).wait()
    plsc.subcore_barrier()

    def _h1(k, _):
        p = k % 2
        base = sub * H1R + k * BLK
        pltpu.sync_copy(src128_hbm.at[pl.ds(base, BLK)], sblk1.at[p])

        @pl.when(k >= 1)
        def _drain():
            for _i in range(BLK):
                pltpu.make_async_copy(ones_t, hist_s_sh.at[sblk1.at[0, 0]],
                                      hsem).wait()

        for i in range(BLK):
            pltpu.async_copy(ones_t, hist_s_sh.at[sblk1.at[p, i]],
                             hsem, add=True)
        return _
    lax.fori_loop(0, H1R // BLK, _h1, None)
    for _i in range(BLK):
        pltpu.make_async_copy(ones_t, hist_s_sh.at[sblk1.at[0, 0]],
                              hsem).wait()
    plsc.subcore_barrier()

    def _norm(ref):
        def _n(g, _):
            d = jnp.maximum(ref[pl.ds(g * L, L)], 1.0)
            ref[pl.ds(g * L, L)] = _rsqrt_newton(d)
            return _
        lax.fori_loop(0, ROWS_PER_TILE // L, _n, None)

    pltpu.sync_copy(hist_s_sh.at[pl.ds(row0, ROWS_PER_TILE)], nd_t)
    _norm(nd_t)
    pltpu.sync_copy(nd_t, hist_s_sh.at[pl.ds(row0, ROWS_PER_TILE)])
    plsc.subcore_barrier()
    pltpu.sync_copy(hist_s_sh, norm_s_t)

    base3 = (core * NS + sub) * CPT

    def _g_issue(pp, r, buf):
        pltpu.async_copy(feat_hbm.at[sblk.at[pp, r]], rows.at[buf],
                         gsem.at[buf])

    def _g_wait(buf):
        pltpu.make_async_copy(feat_hbm.at[sblk.at[0, 0]], rows.at[buf],
                              gsem.at[buf]).wait()

    def _s_issue(pp, r, buf):
        pltpu.async_copy(rows.at[buf], acc_sh.at[dblk.at[pp, r]],
                         ssem.at[buf], add=True)
        pltpu.async_copy(ones_c, hist_d_sh.at[dblk.at[pp, r]],
                         ssem.at[buf], add=True)

    def _s_wait(buf):
        pltpu.make_async_copy(rows.at[buf], acc_sh.at[dblk.at[0, 0]],
                              ssem.at[buf]).wait()
        pltpu.make_async_copy(ones_c, hist_d_sh.at[dblk.at[0, 0]],
                              ssem.at[buf]).wait()

    pltpu.sync_copy(src_hbm.at[pl.ds(base3, BLK)], sblk.at[0])
    pltpu.sync_copy(dst_hbm.at[pl.ds(base3, BLK)], dblk.at[0])
    pltpu.sync_copy(ew_hbm.at[pl.ds(base3, BLK)], eblk.at[0])
    pltpu.sync_copy(src_hbm.at[pl.ds(base3 + BLK, BLK)], sblk.at[1])
    pltpu.sync_copy(dst_hbm.at[pl.ds(base3 + BLK, BLK)], dblk.at[1])
    pltpu.sync_copy(ew_hbm.at[pl.ds(base3 + BLK, BLK)], eblk.at[1])
    for i in range(NBUF - 1):
        _g_issue(0, i, i)

    def _b3(b, _):
        p = b % 2
        for i in range(BLK):
            buf = i % NBUF
            _g_wait(buf)
            for q in range(CHUNK // L):
                sv = sblk.at[p, i][pl.ds(q * L, L)]
                ev = eblk.at[p, i][pl.ds(q * L, L)]
                w_t[pl.ds(q * L, L)] = ev * plsc.load_gather(norm_s_t, [sv])

            def _scale(r2, _c):
                for u in range(2):
                    r = 2 * r2 + u
                    w = _bcast(w_t, r)
                    rv = rows.at[buf, r]
                    for q in range(F // L):
                        rv[pl.ds(q * L, L)] = rv[pl.ds(q * L, L)] * w
                return _c
            lax.fori_loop(0, CHUNK // 2, _scale, None)
            _s_issue(p, i, buf)

            pbuf = (i - 1) % NBUF
            if i == 0:
                @pl.when(b > 0)
                def _w0():
                    _s_wait(pbuf)
            else:
                _s_wait(pbuf)
            if i == 5:
                @pl.when(b + 1 < NBLK)
                def _pref():
                    nb = base3 + (b + 1) * BLK
                    pltpu.sync_copy(src_hbm.at[pl.ds(nb, BLK)],
                                    sblk.at[1 - p])
                    pltpu.sync_copy(dst_hbm.at[pl.ds(nb, BLK)],
                                    dblk.at[1 - p])
                    pltpu.sync_copy(ew_hbm.at[pl.ds(nb, BLK)],
                                    eblk.at[1 - p])
            nxt = b * BLK + i + NBUF - 1

            @pl.when(nxt < CPT)
            def _gi():
                if i + NBUF - 1 < BLK:
                    _g_issue(p, i + NBUF - 1, pbuf)
                else:
                    _g_issue(1 - p, i + NBUF - 1 - BLK, pbuf)
        return _
    lax.fori_loop(0, NBLK, _b3, None)
    _s_wait((CPT - 1) % NBUF)
    plsc.subcore_barrier()

    pltpu.sync_copy(acc_sh.at[pl.ds(row0, ROWS_PER_TILE)],
                    acc_out.at[core, pl.ds(row0, ROWS_PER_TILE)])
    pltpu.sync_copy(hist_d_sh.at[pl.ds(row0, ROWS_PER_TILE)],
                    dh_out.at[core, pl.ds(row0, ROWS_PER_TILE)])


def _sc_aggregate(src_p, dst_p, ew_p, feat_p, src128):
    mesh = plsc.VectorSubcoreMesh(core_axis_name="c", subcore_axis_name="s")
    return pl.kernel(
        _sc_body,
        out_type=[
            jax.ShapeDtypeStruct((NC, N_PAD, F), jnp.float32),
            jax.ShapeDtypeStruct((NC, N_PAD), jnp.float32),
        ],
        mesh=mesh,
        compiler_params=pltpu.CompilerParams(needs_layout_passes=False),
        scratch_types=[
            pltpu.VMEM_SHARED((N_PAD, F), jnp.float32),
            pltpu.VMEM_SHARED((N_PAD,), jnp.float32),
            pltpu.VMEM_SHARED((N_PAD,), jnp.float32),
            pltpu.VMEM((2, BLK, CHUNK), jnp.int32),
            pltpu.VMEM((2, BLK, CHUNK), jnp.int32),
            pltpu.VMEM((2, BLK, CHUNK), jnp.float32),
            pltpu.VMEM((2, BLK, 128), jnp.int32),
            pltpu.VMEM((128,), jnp.float32),
            pltpu.VMEM((CHUNK,), jnp.float32),
            pltpu.VMEM((N_PAD,), jnp.float32),
            pltpu.VMEM((ROWS_PER_TILE,), jnp.float32),
            pltpu.VMEM((NBUF, CHUNK, F), jnp.float32),
            pltpu.VMEM((CHUNK,), jnp.float32),
            pltpu.SemaphoreType.DMA((NBUF,)),
            pltpu.SemaphoreType.DMA((NBUF,)),
            pltpu.SemaphoreType.DMA,
        ],
    )(src_p, dst_p, ew_p, feat_p, src128)


def _tc_body(acc_ref, dh_ref, w_ref, b_ref, out_ref):
    p = acc_ref[0] + acc_ref[1]
    y = jnp.dot(p, w_ref[...], preferred_element_type=jnp.float32)
    d = jnp.maximum(dh_ref[0] + dh_ref[1], 1.0)
    out_ref[...] = y * lax.rsqrt(d) + b_ref[...]


def _tc_matmul(acc, dh, W, b2):
    blk = 1000
    grid = (N_NODES // blk,)
    return pl.pallas_call(
        _tc_body,
        grid=grid,
        in_specs=[
            pl.BlockSpec((NC, blk, F), lambda i: (0, i, 0)),
            pl.BlockSpec((NC, blk, 1), lambda i: (0, i, 0)),
            pl.BlockSpec((F, F), lambda i: (0, 0)),
            pl.BlockSpec((1, F), lambda i: (0, 0)),
        ],
        out_specs=pl.BlockSpec((blk, F), lambda i: (i, 0)),
        out_shape=jax.ShapeDtypeStruct((N_NODES, F), jnp.float32),
    )(acc, dh, W, b2)


@jax.jit
def kernel(feat, edge_index, edge_weight, W, b):
    src = edge_index[0].astype(jnp.int32)
    dst = edge_index[1].astype(jnp.int32)
    e = src.shape[0]
    npad = E_PAD - e
    pad_idx = (jnp.arange(npad, dtype=jnp.int32) % (N_PAD - N_NODES)) + N_NODES
    src_flat = jnp.concatenate([src, pad_idx])
    src_p = src_flat.reshape(E_PAD // CHUNK, CHUNK)
    src128 = src_flat.reshape(E_PAD // 128, 128)
    dst_p = jnp.concatenate([dst, pad_idx]).reshape(E_PAD // CHUNK, CHUNK)
    ew_p = jnp.concatenate(
        [edge_weight, jnp.zeros((npad,), jnp.float32)]
    ).reshape(E_PAD // CHUNK, CHUNK)
    feat_p = jnp.pad(feat, ((0, N_PAD - feat.shape[0]), (0, 0)))
    acc, dh = _sc_aggregate(src_p, dst_p, ew_p, feat_p, src128)
    return _tc_matmul(acc, dh.reshape(NC, N_PAD, 1), W, b.reshape(1, F))
